# Initial kernel scaffold; baseline (speedup 1.0000x reference)
import jax
import jax.numpy as jnp
from jax.experimental import pallas as pl


def kernel(x, edge_index, edge_attr, W_nenc, b_nenc, W_eenc, b_eenc, W_msg, W_eme, W_upd, W_self, W_eup, W_dec1, b_dec1, W_dec2, b_dec2):
    raise NotImplementedError("write your pallas kernel here")


# speedup vs baseline: 1.9246x; 1.9246x over previous
import functools

import jax
import jax.numpy as jnp
from jax import lax
from jax.experimental import pallas as pl
from jax.experimental.pallas import tpu as pltpu
from jax.experimental.pallas import tpu_sc as plsc

N = 10000
E = 320000
B = 4096
DF = 128
DE = 16
H = 64
C = 2

NC, NS = 2, 16
NW = NC * NS
CHUNK = 128
EC = E - B
CPT = -(-EC // (NW * CHUNK))
EP = NW * CPT * CHUNK
PADE = EP - EC
ETOT_P = B + EP
NAGG = 10112
ROWS_PT = NAGG // NS
EPT = CPT * CHUNK

F32 = jnp.float32


def _mesh():
    return plsc.VectorSubcoreMesh(core_axis_name="c", subcore_axis_name="s")


_SC_PARAMS = pltpu.CompilerParams(use_tc_tiling_on_sc=False)


def _node_enc_body(x_ref, wn_ref, bn_ref, wm_ref, h_ref, hm_ref):
    h = jnp.maximum(
        jnp.dot(x_ref[...], wn_ref[...], preferred_element_type=F32)
        + bn_ref[...], 0.0)
    h_ref[...] = h
    hm_ref[...] = jnp.dot(h, wm_ref[...], preferred_element_type=F32)


def _edge_enc_body(ea_ref, we_ref, be_ref, wa_ref, wb_ref, o2_ref, o3_ref):
    e = jnp.maximum(
        jnp.dot(ea_ref[...], we_ref[...], preferred_element_type=F32)
        + be_ref[...], 0.0)
    o2_ref[...] = jnp.dot(e, wa_ref[...], preferred_element_type=F32)
    o3_ref[...] = jnp.dot(e, wb_ref[...], preferred_element_type=F32)


def _node_upd0_body(p_ref, h_ref, wu_ref, ws_ref, w1_ref, w2_ref, wm_ref,
                    h1_ref, u1_ref, u2_ref, hm_ref):
    p = p_ref[...]
    agg = p[0] + p[1]
    h1 = jnp.maximum(
        jnp.dot(agg, wu_ref[...], preferred_element_type=F32)
        + jnp.dot(h_ref[...], ws_ref[...], preferred_element_type=F32), 0.0)
    h1_ref[...] = h1
    u1_ref[...] = jnp.dot(h1, w1_ref[...], preferred_element_type=F32)
    u2_ref[...] = jnp.dot(h1, w2_ref[...], preferred_element_type=F32)
    hm_ref[...] = jnp.dot(h1, wm_ref[...], preferred_element_type=F32)


def _node_upd1_body(p_ref, h_ref, wu_ref, ws_ref, w1_ref, w2_ref, wd1_ref,
                    wd2_ref, u1_ref, u2_ref, d1_ref, d2_ref):
    p = p_ref[...]
    agg = p[0] + p[1]
    h2 = jnp.maximum(
        jnp.dot(agg, wu_ref[...], preferred_element_type=F32)
        + jnp.dot(h_ref[...], ws_ref[...], preferred_element_type=F32), 0.0)
    u1_ref[...] = jnp.dot(h2, w1_ref[...], preferred_element_type=F32)
    u2_ref[...] = jnp.dot(h2, w2_ref[...], preferred_element_type=F32)
    d1_ref[...] = jnp.dot(h2, wd1_ref[...], preferred_element_type=F32)
    d2_ref[...] = jnp.dot(h2, wd2_ref[...], preferred_element_type=F32)


def _e2_body(e_ref, w_ref, o_ref):
    o_ref[...] = jnp.dot(e_ref[...], w_ref[...], preferred_element_type=F32)


def _head_body(te_ref, g_ref, w31_ref, wd3_ref, b1_ref, w2_ref, b2_ref,
               o_ref):
    g = g_ref[...]
    te2 = jnp.maximum(
        g[0] + g[1]
        + jnp.dot(te_ref[...], w31_ref[...], preferred_element_type=F32), 0.0)
    z = jnp.maximum(
        g[2] + g[3]
        + jnp.dot(te2, wd3_ref[...], preferred_element_type=F32)
        + b1_ref[...], 0.0)
    o_ref[...] = jnp.dot(z, w2_ref[...], preferred_element_type=F32) + b2_ref[...]


def _w64(i):
    return pl.BlockSpec((H, H), lambda *_: (0, 0))


def _b64():
    return pl.BlockSpec((1, H), lambda *_: (0, 0))


def _relu_add2(dst, other):
    def row(r, _):
        for cc in range(H // 16):
            sl = pl.ds(cc * 16, 16)
            dst[r, sl] = jnp.maximum(dst[r, sl] + other[r, sl], 0.0)
        return 0
    lax.fori_loop(0, CHUNK, row, 0)


def _relu_add3(dst, o1, o2):
    def row(r, _):
        for cc in range(H // 16):
            sl = pl.ds(cc * 16, 16)
            dst[r, sl] = jnp.maximum(dst[r, sl] + o1[r, sl] + o2[r, sl], 0.0)
        return 0
    lax.fori_loop(0, CHUNK, row, 0)


def _sc_msg_body(hm_ref, e_ref, si_ref, di_ref, z_ref, out_ref,
                 sidx, didx, ebuf, gbuf, zbuf, agg, sem, *, row_off):
    c = lax.axis_index("c")
    s = lax.axis_index("s")
    wid = c * NS + s
    pltpu.sync_copy(si_ref.at[pl.ds(pl.multiple_of(wid * EPT, 8), EPT)], sidx)
    pltpu.sync_copy(di_ref.at[wid], didx)
    rbase = pl.multiple_of(s * ROWS_PT, 8)
    pltpu.sync_copy(z_ref, zbuf)
    pltpu.sync_copy(zbuf, agg.at[pl.ds(rbase, ROWS_PT)])
    plsc.subcore_barrier()
    def chunk(j, _):
        base = pl.multiple_of(row_off + wid * EPT + j * CHUNK, 8)
        pltpu.sync_copy(e_ref.at[pl.ds(base, CHUNK)], ebuf)
        pltpu.async_copy(
            hm_ref.at[sidx.at[pl.ds(pl.multiple_of(j * CHUNK, 8), CHUNK)]],
            gbuf, sem).wait()
        _relu_add2(gbuf, ebuf)
        pltpu.sync_copy(gbuf, agg.at[didx.at[j]], add=True)
        return 0
    lax.fori_loop(0, CPT, chunk, 0)
    plsc.subcore_barrier()
    pltpu.sync_copy(agg.at[pl.ds(rbase, ROWS_PT)], zbuf)
    pltpu.sync_copy(zbuf, out_ref.at[c, pl.ds(rbase, ROWS_PT)])


def _sc_msg_pass(hm, etmp2, sidxf, didx3, zeros_pt, row_off):
    kfn = pl.kernel(
        functools.partial(_sc_msg_body, row_off=row_off),
        out_type=jax.ShapeDtypeStruct((NC, NAGG, H), F32),
        mesh=_mesh(),
        compiler_params=_SC_PARAMS,
        scratch_types=[
            pltpu.VMEM((EPT,), jnp.int32),
            pltpu.VMEM((CPT, CHUNK), jnp.int32),
            pltpu.VMEM((CHUNK, H), F32),
            pltpu.VMEM((CHUNK, H), F32),
            pltpu.VMEM((ROWS_PT, H), F32),
            pltpu.VMEM_SHARED((NAGG, H), F32),
            pltpu.SemaphoreType.DMA,
        ],
    )
    return kfn(hm, etmp2, sidxf, didx3, zeros_pt)


def _sc_eupd_body(u1_ref, u2_ref, e3_ref, si_ref, di_ref, tsi_ref, tdi_ref,
                  e1_ref, te1_ref, sidx, didx, tsid, tdid, ebuf, g1, g2, sem):
    c = lax.axis_index("c")
    s = lax.axis_index("s")
    wid = c * NS + s
    pltpu.sync_copy(si_ref.at[pl.ds(pl.multiple_of(wid * EPT, 8), EPT)], sidx)
    pltpu.sync_copy(di_ref.at[pl.ds(pl.multiple_of(wid * EPT, 8), EPT)], didx)
    def chunk(j, _):
        src_base = pl.multiple_of(B + wid * EPT + j * CHUNK, 8)
        dst_base = pl.multiple_of(wid * EPT + j * CHUNK, 8)
        jsl = pl.ds(pl.multiple_of(j * CHUNK, 8), CHUNK)
        pltpu.sync_copy(e3_ref.at[pl.ds(src_base, CHUNK)], ebuf)
        pltpu.async_copy(u1_ref.at[sidx.at[jsl]], g1, sem).wait()
        pltpu.async_copy(u2_ref.at[didx.at[jsl]], g2, sem).wait()
        _relu_add3(g1, g2, ebuf)
        pltpu.sync_copy(g1, e1_ref.at[pl.ds(dst_base, CHUNK)])
        return 0
    lax.fori_loop(0, CPT, chunk, 0)
    tbase = pl.multiple_of(wid * CHUNK, 8)
    pltpu.sync_copy(tsi_ref.at[pl.ds(tbase, CHUNK)], tsid)
    pltpu.sync_copy(tdi_ref.at[pl.ds(tbase, CHUNK)], tdid)
    pltpu.sync_copy(e3_ref.at[pl.ds(tbase, CHUNK)], ebuf)
    pltpu.async_copy(u1_ref.at[tsid], g1, sem).wait()
    pltpu.async_copy(u2_ref.at[tdid], g2, sem).wait()
    _relu_add3(g1, g2, ebuf)
    pltpu.sync_copy(g1, te1_ref.at[pl.ds(tbase, CHUNK)])


def _sc_eupd_pass(u1, u2, etmp3, sidxf, didxf, tsif, tdif):
    kfn = pl.kernel(
        _sc_eupd_body,
        out_type=(jax.ShapeDtypeStruct((EP, H), F32),
                  jax.ShapeDtypeStruct((B, H), F32)),
        mesh=_mesh(),
        compiler_params=_SC_PARAMS,
        scratch_types=[
            pltpu.VMEM((EPT,), jnp.int32),
            pltpu.VMEM((EPT,), jnp.int32),
            pltpu.VMEM((CHUNK,), jnp.int32),
            pltpu.VMEM((CHUNK,), jnp.int32),
            pltpu.VMEM((CHUNK, H), F32),
            pltpu.VMEM((CHUNK, H), F32),
            pltpu.VMEM((CHUNK, H), F32),
            pltpu.SemaphoreType.DMA,
        ],
    )
    return kfn(u1, u2, etmp3, sidxf, didxf, tsif, tdif)


def _sc_tgather_body(t1_ref, t2_ref, t3_ref, t4_ref, tsi_ref, tdi_ref,
                     out_ref, tsid, tdid, gbuf, sem):
    c = lax.axis_index("c")
    s = lax.axis_index("s")
    wid = c * NS + s
    tbase = pl.multiple_of(wid * CHUNK, 8)
    pltpu.sync_copy(tsi_ref.at[pl.ds(tbase, CHUNK)], tsid)
    pltpu.sync_copy(tdi_ref.at[pl.ds(tbase, CHUNK)], tdid)
    pltpu.async_copy(t1_ref.at[tsid], gbuf, sem).wait()
    pltpu.sync_copy(gbuf, out_ref.at[0, pl.ds(tbase, CHUNK)])
    pltpu.async_copy(t2_ref.at[tdid], gbuf, sem).wait()
    pltpu.sync_copy(gbuf, out_ref.at[1, pl.ds(tbase, CHUNK)])
    pltpu.async_copy(t3_ref.at[tsid], gbuf, sem).wait()
    pltpu.sync_copy(gbuf, out_ref.at[2, pl.ds(tbase, CHUNK)])
    pltpu.async_copy(t4_ref.at[tdid], gbuf, sem).wait()
    pltpu.sync_copy(gbuf, out_ref.at[3, pl.ds(tbase, CHUNK)])


def _sc_tgather(t1, t2, t3, t4, tsif, tdif):
    kfn = pl.kernel(
        _sc_tgather_body,
        out_type=jax.ShapeDtypeStruct((4, B, H), F32),
        mesh=_mesh(),
        compiler_params=_SC_PARAMS,
        scratch_types=[
            pltpu.VMEM((CHUNK,), jnp.int32),
            pltpu.VMEM((CHUNK,), jnp.int32),
            pltpu.VMEM((CHUNK, H), F32),
            pltpu.SemaphoreType.DMA,
        ],
    )
    return kfn(t1, t2, t3, t4, tsif, tdif)


def kernel(x, edge_index, edge_attr, W_nenc, b_nenc, W_eenc, b_eenc,
           W_msg, W_eme, W_upd, W_self, W_eup, W_dec1, b_dec1, W_dec2,
           b_dec2):
    src, dst = edge_index[0, B:], edge_index[1, B:]
    tsrc, tdst = edge_index[0, :B], edge_index[1, :B]

    sidxf = jnp.concatenate([src, jnp.zeros((PADE,), jnp.int32)])
    didxf = jnp.concatenate([dst, jnp.full((PADE,), N, jnp.int32)])
    didx3 = didxf.reshape(NW, CPT, CHUNK)
    eap = jnp.concatenate(
        [edge_attr, jnp.zeros((ETOT_P - E, DE), F32)])
    zeros_pt = jnp.zeros((ROWS_PT, H), F32)

    bn = b_nenc.reshape(1, H)
    be = b_eenc.reshape(1, H)
    b1 = b_dec1.reshape(1, H)
    w2p = jnp.pad(W_dec2, ((0, 0), (0, 128 - C)))
    b2p = jnp.pad(b_dec2, (0, 128 - C)).reshape(1, 128)

    we0_1, we0_2, we0_3 = W_eup[0][:H], W_eup[0][H:2 * H], W_eup[0][2 * H:]
    we1_1, we1_2, we1_3 = W_eup[1][:H], W_eup[1][H:2 * H], W_eup[1][2 * H:]
    wd1_1, wd1_2, wd1_3 = W_dec1[:H], W_dec1[H:2 * H], W_dec1[2 * H:]

    NBLK = 2000
    h, hm0 = pl.pallas_call(
        _node_enc_body,
        grid=(N // NBLK,),
        in_specs=[
            pl.BlockSpec((NBLK, DF), lambda i: (i, 0)),
            pl.BlockSpec((DF, H), lambda i: (0, 0)),
            _b64(), _w64(0),
        ],
        out_specs=[pl.BlockSpec((NBLK, H), lambda i: (i, 0))] * 2,
        out_shape=[jax.ShapeDtypeStruct((N, H), F32)] * 2,
    )(x, W_nenc, bn, W_msg[0])

    EBLK = 4096
    etmp2_0, etmp3_0 = pl.pallas_call(
        _edge_enc_body,
        grid=(ETOT_P // EBLK,),
        in_specs=[
            pl.BlockSpec((EBLK, DE), lambda i: (i, 0)),
            pl.BlockSpec((DE, H), lambda i: (0, 0)),
            _b64(), _w64(0), _w64(1),
        ],
        out_specs=[pl.BlockSpec((EBLK, H), lambda i: (i, 0))] * 2,
        out_shape=[jax.ShapeDtypeStruct((ETOT_P, H), F32)] * 2,
    )(eap, W_eenc, be, W_eme[0], we0_3)

    part0 = _sc_msg_pass(hm0, etmp2_0, sidxf, didx3, zeros_pt, row_off=B)

    h1, hu1_0, hu2_0, hm1 = pl.pallas_call(
        _node_upd0_body,
        grid=(N // NBLK,),
        in_specs=[
            pl.BlockSpec((NC, NBLK, H), lambda i: (0, i, 0)),
            pl.BlockSpec((NBLK, H), lambda i: (i, 0)),
            _w64(0), _w64(1), _w64(2), _w64(3), _w64(4),
        ],
        out_specs=[pl.BlockSpec((NBLK, H), lambda i: (i, 0))] * 4,
        out_shape=[jax.ShapeDtypeStruct((N, H), F32)] * 4,
    )(part0, h, W_upd[0], W_self[0], we0_1, we0_2, W_msg[1])

    e1, te1 = _sc_eupd_pass(hu1_0, hu2_0, etmp3_0, sidxf, didxf, tsrc, tdst)

    etmp2_1 = pl.pallas_call(
        _e2_body,
        grid=(EP // EBLK,),
        in_specs=[
            pl.BlockSpec((EBLK, H), lambda i: (i, 0)),
            _w64(0),
        ],
        out_specs=pl.BlockSpec((EBLK, H), lambda i: (i, 0)),
        out_shape=jax.ShapeDtypeStruct((EP, H), F32),
    )(e1, W_eme[1])

    part1 = _sc_msg_pass(hm1, etmp2_1, sidxf, didx3, zeros_pt, row_off=0)

    hu1_1, hu2_1, hd1, hd2 = pl.pallas_call(
        _node_upd1_body,
        grid=(N // NBLK,),
        in_specs=[
            pl.BlockSpec((NC, NBLK, H), lambda i: (0, i, 0)),
            pl.BlockSpec((NBLK, H), lambda i: (i, 0)),
            _w64(0), _w64(1), _w64(2), _w64(3), _w64(4), _w64(5),
        ],
        out_specs=[pl.BlockSpec((NBLK, H), lambda i: (i, 0))] * 4,
        out_shape=[jax.ShapeDtypeStruct((N, H), F32)] * 4,
    )(part1, h1, W_upd[1], W_self[1], we1_1, we1_2, wd1_1, wd1_2)

    g = _sc_tgather(hu1_1, hu2_1, hd1, hd2, tsrc, tdst)

    TBLK = 1024
    outp = pl.pallas_call(
        _head_body,
        grid=(B // TBLK,),
        in_specs=[
            pl.BlockSpec((TBLK, H), lambda i: (i, 0)),
            pl.BlockSpec((4, TBLK, H), lambda i: (0, i, 0)),
            _w64(0), _w64(1), _b64(),
            pl.BlockSpec((H, 128), lambda i: (0, 0)),
            pl.BlockSpec((1, 128), lambda i: (0, 0)),
        ],
        out_specs=pl.BlockSpec((TBLK, 128), lambda i: (i, 0)),
        out_shape=jax.ShapeDtypeStruct((B, 128), F32),
    )(te1, g, we1_3, wd1_3, b1, w2p, b2p)

    return outp[:, :C]


# speedup vs baseline: 2.6152x; 1.3588x over previous
import functools

import jax
import jax.numpy as jnp
from jax import lax
from jax.experimental import pallas as pl
from jax.experimental.pallas import tpu as pltpu
from jax.experimental.pallas import tpu_sc as plsc

N = 10000
E = 320000
B = 4096
DF = 128
DE = 16
H = 64
C = 2

NC, NS = 2, 16
NW = NC * NS
CHUNK = 128
EC = E - B
CPT = -(-EC // (NW * CHUNK))
EP = NW * CPT * CHUNK
PADE = EP - EC
ETOT_P = B + EP
NAGG = 10112
ROWS_PT = NAGG // NS
EPT = CPT * CHUNK

F32 = jnp.float32


def _mesh():
    return plsc.VectorSubcoreMesh(core_axis_name="c", subcore_axis_name="s")


_SC_PARAMS = pltpu.CompilerParams(use_tc_tiling_on_sc=False)


def _node_enc_body(x_ref, wn_ref, bn_ref, wm_ref, h_ref, hm_ref):
    h = jnp.maximum(
        jnp.dot(x_ref[...], wn_ref[...], preferred_element_type=F32)
        + bn_ref[...], 0.0)
    h_ref[...] = h
    hm_ref[...] = jnp.dot(h, wm_ref[...], preferred_element_type=F32)


def _edge_enc_body(ea_ref, we_ref, be_ref, wa_ref, wb_ref, o2_ref, o3_ref):
    e = jnp.maximum(
        jnp.dot(ea_ref[...], we_ref[...], preferred_element_type=F32)
        + be_ref[...], 0.0)
    o2_ref[...] = jnp.dot(e, wa_ref[...], preferred_element_type=F32)
    o3_ref[...] = jnp.dot(e, wb_ref[...], preferred_element_type=F32)


def _node_upd0_body(p_ref, h_ref, wu_ref, ws_ref, w1_ref, w2_ref, wm_ref,
                    h1_ref, u1_ref, u2_ref, hm_ref):
    p = p_ref[...]
    agg = p[0] + p[1]
    h1 = jnp.maximum(
        jnp.dot(agg, wu_ref[...], preferred_element_type=F32)
        + jnp.dot(h_ref[...], ws_ref[...], preferred_element_type=F32), 0.0)
    h1_ref[...] = h1
    u1_ref[...] = jnp.dot(h1, w1_ref[...], preferred_element_type=F32)
    u2_ref[...] = jnp.dot(h1, w2_ref[...], preferred_element_type=F32)
    hm_ref[...] = jnp.dot(h1, wm_ref[...], preferred_element_type=F32)


def _node_upd1_body(p_ref, h_ref, wu_ref, ws_ref, w1_ref, w2_ref, wd1_ref,
                    wd2_ref, u1_ref, u2_ref, d1_ref, d2_ref):
    p = p_ref[...]
    agg = p[0] + p[1]
    h2 = jnp.maximum(
        jnp.dot(agg, wu_ref[...], preferred_element_type=F32)
        + jnp.dot(h_ref[...], ws_ref[...], preferred_element_type=F32), 0.0)
    u1_ref[...] = jnp.dot(h2, w1_ref[...], preferred_element_type=F32)
    u2_ref[...] = jnp.dot(h2, w2_ref[...], preferred_element_type=F32)
    d1_ref[...] = jnp.dot(h2, wd1_ref[...], preferred_element_type=F32)
    d2_ref[...] = jnp.dot(h2, wd2_ref[...], preferred_element_type=F32)


def _e2_body(e_ref, w_ref, o_ref):
    o_ref[...] = jnp.dot(e_ref[...], w_ref[...], preferred_element_type=F32)


def _head_body(te_ref, g_ref, w31_ref, wd3_ref, b1_ref, w2_ref, b2_ref,
               o_ref):
    g = g_ref[...]
    te2 = jnp.maximum(
        g[0] + g[1]
        + jnp.dot(te_ref[...], w31_ref[...], preferred_element_type=F32), 0.0)
    z = jnp.maximum(
        g[2] + g[3]
        + jnp.dot(te2, wd3_ref[...], preferred_element_type=F32)
        + b1_ref[...], 0.0)
    o_ref[...] = jnp.dot(z, w2_ref[...], preferred_element_type=F32) + b2_ref[...]


def _w64(i):
    return pl.BlockSpec((H, H), lambda *_: (0, 0))


def _b64():
    return pl.BlockSpec((1, H), lambda *_: (0, 0))


def _relu_add2(dst, other):
    def row(r, _):
        for cc in range(H // 16):
            sl = pl.ds(cc * 16, 16)
            dst[r, sl] = jnp.maximum(dst[r, sl] + other[r, sl], 0.0)
        return 0
    lax.fori_loop(0, CHUNK, row, 0)


def _relu_add3(dst, o1, o2):
    def row(r, _):
        for cc in range(H // 16):
            sl = pl.ds(cc * 16, 16)
            dst[r, sl] = jnp.maximum(dst[r, sl] + o1[r, sl] + o2[r, sl], 0.0)
        return 0
    lax.fori_loop(0, CHUNK, row, 0)


MR = 4
ER = 4
ZCH = (CHUNK, CHUNK, CHUNK, CHUNK, ROWS_PT - 4 * CHUNK)


def _sc_msg_body(hm_ref, e_ref, si_ref, di_ref, z_ref, out_ref, *refs,
                 row_off):
    sidx, didx, agg = refs[0], refs[1], refs[2]
    eb = refs[3:3 + MR]
    gb = refs[3 + MR:3 + 2 * MR]
    sl = refs[3 + 2 * MR:3 + 3 * MR]
    sg = refs[3 + 3 * MR:3 + 4 * MR]
    st = refs[3 + 4 * MR:3 + 5 * MR]
    c = lax.axis_index("c")
    s = lax.axis_index("s")
    wid = c * NS + s
    ebase = row_off + wid * EPT

    def ld_start(u, b):
        pltpu.async_copy(
            e_ref.at[pl.ds(pl.multiple_of(ebase + u * CHUNK, 8), CHUNK)],
            eb[b], sl[b])

    def ld_wait(u, b):
        pltpu.make_async_copy(
            e_ref.at[pl.ds(pl.multiple_of(ebase + u * CHUNK, 8), CHUNK)],
            eb[b], sl[b]).wait()

    def g_start(u, b):
        pltpu.async_copy(
            hm_ref.at[sidx.at[pl.ds(pl.multiple_of(u * CHUNK, 8), CHUNK)]],
            gb[b], sg[b])

    def g_wait(u, b):
        pltpu.make_async_copy(
            hm_ref.at[sidx.at[pl.ds(pl.multiple_of(u * CHUNK, 8), CHUNK)]],
            gb[b], sg[b]).wait()

    def st_start(u, b):
        pltpu.async_copy(gb[b], agg.at[didx.at[u]], st[b], add=True)

    def st_wait(u, b):
        pltpu.make_async_copy(gb[b], agg.at[didx.at[u]], st[b]).wait()

    rbase = pl.multiple_of(s * ROWS_PT, 8)
    pltpu.sync_copy(z_ref, eb[0])
    off = 0
    for n in ZCH:
        pltpu.sync_copy(eb[0].at[pl.ds(0, n)],
                        agg.at[pl.ds(rbase + off, n)])
        off += n
    plsc.subcore_barrier()

    pltpu.sync_copy(si_ref.at[pl.ds(pl.multiple_of(wid * EPT, 8), EPT)], sidx)
    pltpu.sync_copy(di_ref.at[wid], didx)

    for u in range(2):
        ld_start(u, u)
        g_start(u, u)
    for k in range(4):
        u = k
        bA = (k + 2) % 4
        if u >= 2:
            st_wait(u - 2, bA)
        ld_start(u + 2, bA)
        g_start(u + 2, bA)
        ld_wait(u, k)
        g_wait(u, k)
        _relu_add2(gb[k], eb[k])
        st_start(u, k)

    def grp(t, _):
        for k in range(4):
            u = t * 4 + k
            bA = (k + 2) % 4
            st_wait(u - 2, bA)
            ld_start(u + 2, bA)
            g_start(u + 2, bA)
            ld_wait(u, k)
            g_wait(u, k)
            _relu_add2(gb[k], eb[k])
            st_start(u, k)
        return 0
    lax.fori_loop(1, 17, grp, 0)

    for u in range(68, 78):
        b = u % 4
        if u + 2 <= 77:
            bA = (u + 2) % 4
            st_wait(u - 2, bA)
            ld_start(u + 2, bA)
            g_start(u + 2, bA)
        ld_wait(u, b)
        g_wait(u, b)
        _relu_add2(gb[b], eb[b])
        st_start(u, b)
    for u in range(76, 78):
        st_wait(u, u % 4)

    plsc.subcore_barrier()
    off = 0
    for n in ZCH:
        pltpu.sync_copy(agg.at[pl.ds(rbase + off, n)], eb[0].at[pl.ds(0, n)])
        pltpu.sync_copy(eb[0].at[pl.ds(0, n)],
                        out_ref.at[c, pl.ds(rbase + off, n)])
        off += n


def _sc_msg_pass(hm, etmp2, sidxf, didx3, zeros_pt, row_off):
    kfn = pl.kernel(
        functools.partial(_sc_msg_body, row_off=row_off),
        out_type=jax.ShapeDtypeStruct((NC, NAGG, H), F32),
        mesh=_mesh(),
        compiler_params=_SC_PARAMS,
        scratch_types=[
            pltpu.VMEM((EPT,), jnp.int32),
            pltpu.VMEM((CPT, CHUNK), jnp.int32),
            pltpu.VMEM_SHARED((NAGG, H), F32),
        ] + [pltpu.VMEM((CHUNK, H), F32)] * (2 * MR)
          + [pltpu.SemaphoreType.DMA] * (3 * MR),
    )
    return kfn(hm, etmp2, sidxf, didx3, zeros_pt)


def _sc_eupd_body(u1_ref, u2_ref, e3_ref, si_ref, di_ref, tsi_ref, tdi_ref,
                  e1_ref, te1_ref, *refs):
    sidx, didx, tsid, tdid = refs[0], refs[1], refs[2], refs[3]
    eb = refs[4:4 + ER]
    g1 = refs[4 + ER:4 + 2 * ER]
    g2 = refs[4 + 2 * ER:4 + 3 * ER]
    sl = refs[4 + 3 * ER:4 + 4 * ER]
    sg = refs[4 + 4 * ER:4 + 5 * ER]
    st = refs[4 + 5 * ER:4 + 6 * ER]
    c = lax.axis_index("c")
    s = lax.axis_index("s")
    wid = c * NS + s

    def ld_start(u, b):
        pltpu.async_copy(
            e3_ref.at[pl.ds(pl.multiple_of(B + wid * EPT + u * CHUNK, 8),
                            CHUNK)], eb[b], sl[b])

    def ld_wait(u, b):
        pltpu.make_async_copy(
            e3_ref.at[pl.ds(pl.multiple_of(B + wid * EPT + u * CHUNK, 8),
                            CHUNK)], eb[b], sl[b]).wait()

    def g_start(u, b):
        jsl = pl.ds(pl.multiple_of(u * CHUNK, 8), CHUNK)
        pltpu.async_copy(u1_ref.at[sidx.at[jsl]], g1[b], sg[b])
        pltpu.async_copy(u2_ref.at[didx.at[jsl]], g2[b], sg[b])

    def g_wait(u, b):
        jsl = pl.ds(pl.multiple_of(u * CHUNK, 8), CHUNK)
        pltpu.make_async_copy(u1_ref.at[sidx.at[jsl]], g1[b], sg[b]).wait()
        pltpu.make_async_copy(u2_ref.at[didx.at[jsl]], g2[b], sg[b]).wait()

    def st_start(u, b):
        pltpu.async_copy(
            g1[b],
            e1_ref.at[pl.ds(pl.multiple_of(wid * EPT + u * CHUNK, 8), CHUNK)],
            st[b])

    def st_wait(u, b):
        pltpu.make_async_copy(
            g1[b],
            e1_ref.at[pl.ds(pl.multiple_of(wid * EPT + u * CHUNK, 8), CHUNK)],
            st[b]).wait()

    pltpu.sync_copy(si_ref.at[pl.ds(pl.multiple_of(wid * EPT, 8), EPT)], sidx)
    pltpu.sync_copy(di_ref.at[pl.ds(pl.multiple_of(wid * EPT, 8), EPT)], didx)

    for u in range(2):
        ld_start(u, u)
        g_start(u, u)
    for k in range(4):
        u = k
        bA = (k + 2) % 4
        if u >= 2:
            st_wait(u - 2, bA)
        ld_start(u + 2, bA)
        g_start(u + 2, bA)
        ld_wait(u, k)
        g_wait(u, k)
        _relu_add3(g1[k], g2[k], eb[k])
        st_start(u, k)

    def grp(t, _):
        for k in range(4):
            u = t * 4 + k
            bA = (k + 2) % 4
            st_wait(u - 2, bA)
            ld_start(u + 2, bA)
            g_start(u + 2, bA)
            ld_wait(u, k)
            g_wait(u, k)
            _relu_add3(g1[k], g2[k], eb[k])
            st_start(u, k)
        return 0
    lax.fori_loop(1, 17, grp, 0)

    for u in range(68, 78):
        b = u % 4
        if u + 2 <= 77:
            bA = (u + 2) % 4
            st_wait(u - 2, bA)
            ld_start(u + 2, bA)
            g_start(u + 2, bA)
        ld_wait(u, b)
        g_wait(u, b)
        _relu_add3(g1[b], g2[b], eb[b])
        st_start(u, b)
    for u in range(76, 78):
        st_wait(u, u % 4)

    tbase = pl.multiple_of(wid * CHUNK, 8)
    pltpu.sync_copy(tsi_ref.at[pl.ds(tbase, CHUNK)], tsid)
    pltpu.sync_copy(tdi_ref.at[pl.ds(tbase, CHUNK)], tdid)
    pltpu.sync_copy(e3_ref.at[pl.ds(tbase, CHUNK)], eb[0])
    pltpu.async_copy(u1_ref.at[tsid], g1[0], sg[0]).wait()
    pltpu.async_copy(u2_ref.at[tdid], g2[0], sg[0]).wait()
    _relu_add3(g1[0], g2[0], eb[0])
    pltpu.sync_copy(g1[0], te1_ref.at[pl.ds(tbase, CHUNK)])


def _sc_eupd_pass(u1, u2, etmp3, sidxf, didxf, tsif, tdif):
    kfn = pl.kernel(
        _sc_eupd_body,
        out_type=(jax.ShapeDtypeStruct((EP, H), F32),
                  jax.ShapeDtypeStruct((B, H), F32)),
        mesh=_mesh(),
        compiler_params=_SC_PARAMS,
        scratch_types=[
            pltpu.VMEM((EPT,), jnp.int32),
            pltpu.VMEM((EPT,), jnp.int32),
            pltpu.VMEM((CHUNK,), jnp.int32),
            pltpu.VMEM((CHUNK,), jnp.int32),
        ] + [pltpu.VMEM((CHUNK, H), F32)] * (3 * ER)
          + [pltpu.SemaphoreType.DMA] * (3 * ER),
    )
    return kfn(u1, u2, etmp3, sidxf, didxf, tsif, tdif)


def _sc_tgather_body(t1_ref, t2_ref, t3_ref, t4_ref, tsi_ref, tdi_ref,
                     out_ref, tsid, tdid, gbuf, sem):
    c = lax.axis_index("c")
    s = lax.axis_index("s")
    wid = c * NS + s
    tbase = pl.multiple_of(wid * CHUNK, 8)
    pltpu.sync_copy(tsi_ref.at[pl.ds(tbase, CHUNK)], tsid)
    pltpu.sync_copy(tdi_ref.at[pl.ds(tbase, CHUNK)], tdid)
    pltpu.async_copy(t1_ref.at[tsid], gbuf, sem).wait()
    pltpu.sync_copy(gbuf, out_ref.at[0, pl.ds(tbase, CHUNK)])
    pltpu.async_copy(t2_ref.at[tdid], gbuf, sem).wait()
    pltpu.sync_copy(gbuf, out_ref.at[1, pl.ds(tbase, CHUNK)])
    pltpu.async_copy(t3_ref.at[tsid], gbuf, sem).wait()
    pltpu.sync_copy(gbuf, out_ref.at[2, pl.ds(tbase, CHUNK)])
    pltpu.async_copy(t4_ref.at[tdid], gbuf, sem).wait()
    pltpu.sync_copy(gbuf, out_ref.at[3, pl.ds(tbase, CHUNK)])


def _sc_tgather(t1, t2, t3, t4, tsif, tdif):
    kfn = pl.kernel(
        _sc_tgather_body,
        out_type=jax.ShapeDtypeStruct((4, B, H), F32),
        mesh=_mesh(),
        compiler_params=_SC_PARAMS,
        scratch_types=[
            pltpu.VMEM((CHUNK,), jnp.int32),
            pltpu.VMEM((CHUNK,), jnp.int32),
            pltpu.VMEM((CHUNK, H), F32),
            pltpu.SemaphoreType.DMA,
        ],
    )
    return kfn(t1, t2, t3, t4, tsif, tdif)


def kernel(x, edge_index, edge_attr, W_nenc, b_nenc, W_eenc, b_eenc,
           W_msg, W_eme, W_upd, W_self, W_eup, W_dec1, b_dec1, W_dec2,
           b_dec2):
    src, dst = edge_index[0, B:], edge_index[1, B:]
    tsrc, tdst = edge_index[0, :B], edge_index[1, :B]

    sidxf = jnp.concatenate([src, jnp.zeros((PADE,), jnp.int32)])
    didxf = jnp.concatenate([dst, jnp.full((PADE,), N, jnp.int32)])
    didx3 = didxf.reshape(NW, CPT, CHUNK)
    eap = jnp.concatenate(
        [edge_attr, jnp.zeros((ETOT_P - E, DE), F32)])
    zeros_pt = jnp.zeros((CHUNK, H), F32)

    bn = b_nenc.reshape(1, H)
    be = b_eenc.reshape(1, H)
    b1 = b_dec1.reshape(1, H)
    w2p = jnp.pad(W_dec2, ((0, 0), (0, 128 - C)))
    b2p = jnp.pad(b_dec2, (0, 128 - C)).reshape(1, 128)

    we0_1, we0_2, we0_3 = W_eup[0][:H], W_eup[0][H:2 * H], W_eup[0][2 * H:]
    we1_1, we1_2, we1_3 = W_eup[1][:H], W_eup[1][H:2 * H], W_eup[1][2 * H:]
    wd1_1, wd1_2, wd1_3 = W_dec1[:H], W_dec1[H:2 * H], W_dec1[2 * H:]

    NBLK = 2000
    h, hm0 = pl.pallas_call(
        _node_enc_body,
        grid=(N // NBLK,),
        in_specs=[
            pl.BlockSpec((NBLK, DF), lambda i: (i, 0)),
            pl.BlockSpec((DF, H), lambda i: (0, 0)),
            _b64(), _w64(0),
        ],
        out_specs=[pl.BlockSpec((NBLK, H), lambda i: (i, 0))] * 2,
        out_shape=[jax.ShapeDtypeStruct((N, H), F32)] * 2,
    )(x, W_nenc, bn, W_msg[0])

    EBLK = 4096
    etmp2_0, etmp3_0 = pl.pallas_call(
        _edge_enc_body,
        grid=(ETOT_P // EBLK,),
        in_specs=[
            pl.BlockSpec((EBLK, DE), lambda i: (i, 0)),
            pl.BlockSpec((DE, H), lambda i: (0, 0)),
            _b64(), _w64(0), _w64(1),
        ],
        out_specs=[pl.BlockSpec((EBLK, H), lambda i: (i, 0))] * 2,
        out_shape=[jax.ShapeDtypeStruct((ETOT_P, H), F32)] * 2,
    )(eap, W_eenc, be, W_eme[0], we0_3)

    part0 = _sc_msg_pass(hm0, etmp2_0, sidxf, didx3, zeros_pt, row_off=B)

    h1, hu1_0, hu2_0, hm1 = pl.pallas_call(
        _node_upd0_body,
        grid=(N // NBLK,),
        in_specs=[
            pl.BlockSpec((NC, NBLK, H), lambda i: (0, i, 0)),
            pl.BlockSpec((NBLK, H), lambda i: (i, 0)),
            _w64(0), _w64(1), _w64(2), _w64(3), _w64(4),
        ],
        out_specs=[pl.BlockSpec((NBLK, H), lambda i: (i, 0))] * 4,
        out_shape=[jax.ShapeDtypeStruct((N, H), F32)] * 4,
    )(part0, h, W_upd[0], W_self[0], we0_1, we0_2, W_msg[1])

    e1, te1 = _sc_eupd_pass(hu1_0, hu2_0, etmp3_0, sidxf, didxf, tsrc, tdst)

    etmp2_1 = pl.pallas_call(
        _e2_body,
        grid=(EP // EBLK,),
        in_specs=[
            pl.BlockSpec((EBLK, H), lambda i: (i, 0)),
            _w64(0),
        ],
        out_specs=pl.BlockSpec((EBLK, H), lambda i: (i, 0)),
        out_shape=jax.ShapeDtypeStruct((EP, H), F32),
    )(e1, W_eme[1])

    part1 = _sc_msg_pass(hm1, etmp2_1, sidxf, didx3, zeros_pt, row_off=0)

    hu1_1, hu2_1, hd1, hd2 = pl.pallas_call(
        _node_upd1_body,
        grid=(N // NBLK,),
        in_specs=[
            pl.BlockSpec((NC, NBLK, H), lambda i: (0, i, 0)),
            pl.BlockSpec((NBLK, H), lambda i: (i, 0)),
            _w64(0), _w64(1), _w64(2), _w64(3), _w64(4), _w64(5),
        ],
        out_specs=[pl.BlockSpec((NBLK, H), lambda i: (i, 0))] * 4,
        out_shape=[jax.ShapeDtypeStruct((N, H), F32)] * 4,
    )(part1, h1, W_upd[1], W_self[1], we1_1, we1_2, wd1_1, wd1_2)

    g = _sc_tgather(hu1_1, hu2_1, hd1, hd2, tsrc, tdst)

    TBLK = 1024
    outp = pl.pallas_call(
        _head_body,
        grid=(B // TBLK,),
        in_specs=[
            pl.BlockSpec((TBLK, H), lambda i: (i, 0)),
            pl.BlockSpec((4, TBLK, H), lambda i: (0, i, 0)),
            _w64(0), _w64(1), _b64(),
            pl.BlockSpec((H, 128), lambda i: (0, 0)),
            pl.BlockSpec((1, 128), lambda i: (0, 0)),
        ],
        out_specs=pl.BlockSpec((TBLK, 128), lambda i: (i, 0)),
        out_shape=jax.ShapeDtypeStruct((B, 128), F32),
    )(te1, g, we1_3, wd1_3, b1, w2p, b2p)

    return outp[:, :C]


# speedup vs baseline: 3.7506x; 1.4342x over previous
import functools

import jax
import jax.numpy as jnp
from jax import lax
from jax.experimental import pallas as pl
from jax.experimental.pallas import tpu as pltpu
from jax.experimental.pallas import tpu_sc as plsc

N = 10000
E = 320000
B = 4096
DF = 128
DE = 16
H = 64
C = 2

NC, NS = 2, 16
NW = NC * NS
CHUNK = 128
EC = E - B
CPT = -(-EC // (NW * CHUNK))
EP = NW * CPT * CHUNK
PADE = EP - EC
ETOT_P = B + EP
NAGG = 10112
ROWS_PT = NAGG // NS
EPT = CPT * CHUNK
HPT = EPT // 2
PCH = CHUNK // 2

F32 = jnp.float32


def _mesh():
    return plsc.VectorSubcoreMesh(core_axis_name="c", subcore_axis_name="s")


_SC_PARAMS = pltpu.CompilerParams(use_tc_tiling_on_sc=False)


def _node_enc_body(x_ref, wn_ref, bn_ref, wm_ref, h_ref, hm_ref):
    h = jnp.maximum(
        jnp.dot(x_ref[...], wn_ref[...], preferred_element_type=F32)
        + bn_ref[...], 0.0)
    h_ref[...] = h
    hm_ref[...] = jnp.dot(h, wm_ref[...], preferred_element_type=F32)


def _edge_enc_body(ea_ref, we_ref, be_ref, wa_ref, wb_ref, o2_ref, o3_ref):
    a = ea_ref[...]
    w = we_ref[...]
    b = be_ref[...]
    e0 = jnp.maximum(jnp.dot(a[:, :DE], w, preferred_element_type=F32) + b, 0.0)
    e1 = jnp.maximum(jnp.dot(a[:, DE:], w, preferred_element_type=F32) + b, 0.0)
    wa = wa_ref[...]
    wb = wb_ref[...]
    o2_ref[...] = jnp.concatenate(
        [jnp.dot(e0, wa, preferred_element_type=F32),
         jnp.dot(e1, wa, preferred_element_type=F32)], axis=1)
    o3_ref[...] = jnp.concatenate(
        [jnp.dot(e0, wb, preferred_element_type=F32),
         jnp.dot(e1, wb, preferred_element_type=F32)], axis=1)


def _node_upd0_body(p_ref, h_ref, wu_ref, ws_ref, w1_ref, w2_ref, wm_ref,
                    h1_ref, u1_ref, u2_ref, hm_ref):
    p = p_ref[...]
    agg = p[0] + p[1]
    h1 = jnp.maximum(
        jnp.dot(agg, wu_ref[...], preferred_element_type=F32)
        + jnp.dot(h_ref[...], ws_ref[...], preferred_element_type=F32), 0.0)
    h1_ref[...] = h1
    u1_ref[...] = jnp.dot(h1, w1_ref[...], preferred_element_type=F32)
    u2_ref[...] = jnp.dot(h1, w2_ref[...], preferred_element_type=F32)
    hm_ref[...] = jnp.dot(h1, wm_ref[...], preferred_element_type=F32)


def _node_upd1_body(p_ref, h_ref, wu_ref, ws_ref, w1_ref, w2_ref, wd1_ref,
                    wd2_ref, u1_ref, u2_ref, d1_ref, d2_ref):
    p = p_ref[...]
    agg = p[0] + p[1]
    h2 = jnp.maximum(
        jnp.dot(agg, wu_ref[...], preferred_element_type=F32)
        + jnp.dot(h_ref[...], ws_ref[...], preferred_element_type=F32), 0.0)
    u1_ref[...] = jnp.dot(h2, w1_ref[...], preferred_element_type=F32)
    u2_ref[...] = jnp.dot(h2, w2_ref[...], preferred_element_type=F32)
    d1_ref[...] = jnp.dot(h2, wd1_ref[...], preferred_element_type=F32)
    d2_ref[...] = jnp.dot(h2, wd2_ref[...], preferred_element_type=F32)


def _e2_body(e_ref, w_ref, o_ref):
    x = e_ref[...]
    w = w_ref[...]
    o_ref[...] = jnp.concatenate(
        [jnp.dot(x[:, :H], w, preferred_element_type=F32),
         jnp.dot(x[:, H:], w, preferred_element_type=F32)], axis=1)


def _head_body(te_ref, g_ref, w31_ref, wd3_ref, b1_ref, w2_ref, b2_ref,
               o_ref):
    g = g_ref[...]
    te2 = jnp.maximum(
        g[0] + g[1]
        + jnp.dot(te_ref[...], w31_ref[...], preferred_element_type=F32), 0.0)
    z = jnp.maximum(
        g[2] + g[3]
        + jnp.dot(te2, wd3_ref[...], preferred_element_type=F32)
        + b1_ref[...], 0.0)
    o_ref[...] = jnp.dot(z, w2_ref[...], preferred_element_type=F32) + b2_ref[...]


def _w64(i):
    return pl.BlockSpec((H, H), lambda *_: (0, 0))


def _b64():
    return pl.BlockSpec((1, H), lambda *_: (0, 0))


def _relu_add2p(dst, ep):
    def row(rr, _):
        for h2 in range(2):
            for cc in range(H // 16):
                sl = pl.ds(cc * 16, 16)
                sp = pl.ds(h2 * H + cc * 16, 16)
                dst[2 * rr + h2, sl] = jnp.maximum(
                    dst[2 * rr + h2, sl] + ep[rr, sp], 0.0)
        return 0
    lax.fori_loop(0, CHUNK // 2, row, 0)


def _relu_add3p(ep, g1, g2):
    def row(rr, _):
        for h2 in range(2):
            for cc in range(H // 16):
                sl = pl.ds(cc * 16, 16)
                sp = pl.ds(h2 * H + cc * 16, 16)
                ep[rr, sp] = jnp.maximum(
                    g1[2 * rr + h2, sl] + g2[2 * rr + h2, sl] + ep[rr, sp],
                    0.0)
        return 0
    lax.fori_loop(0, CHUNK // 2, row, 0)


def _relu_add3(dst, o1, o2):
    def row(r, _):
        for cc in range(H // 16):
            sl = pl.ds(cc * 16, 16)
            dst[r, sl] = jnp.maximum(dst[r, sl] + o1[r, sl] + o2[r, sl], 0.0)
        return 0
    lax.fori_loop(0, CHUNK, row, 0)


MR = 4
ER = 4
ZCH = (CHUNK, CHUNK, CHUNK, CHUNK, ROWS_PT - 4 * CHUNK)


def _sc_msg_body(hm_ref, e_ref, si_ref, di_ref, z_ref, out_ref, *refs,
                 row_off):
    sidx, didx, agg = refs[0], refs[1], refs[2]
    eb = refs[3:3 + MR]
    gb = refs[3 + MR:3 + 2 * MR]
    sl = refs[3 + 2 * MR:3 + 3 * MR]
    sg = refs[3 + 3 * MR:3 + 4 * MR]
    st = refs[3 + 4 * MR:3 + 5 * MR]
    c = lax.axis_index("c")
    s = lax.axis_index("s")
    wid = c * NS + s
    ebase = row_off + wid * HPT

    def ld_start(u, b):
        pltpu.async_copy(
            e_ref.at[pl.ds(pl.multiple_of(ebase + u * PCH, 8), PCH)],
            eb[b], sl[b])

    def ld_wait(u, b):
        pltpu.make_async_copy(
            e_ref.at[pl.ds(pl.multiple_of(ebase + u * PCH, 8), PCH)],
            eb[b], sl[b]).wait()

    def g_start(u, b):
        pltpu.async_copy(
            hm_ref.at[sidx.at[pl.ds(pl.multiple_of(u * CHUNK, 8), CHUNK)]],
            gb[b], sg[b])

    def g_wait(u, b):
        pltpu.make_async_copy(
            hm_ref.at[sidx.at[pl.ds(pl.multiple_of(u * CHUNK, 8), CHUNK)]],
            gb[b], sg[b]).wait()

    def st_start(u, b):
        pltpu.async_copy(gb[b], agg.at[didx.at[u]], st[b], add=True)

    def st_wait(u, b):
        pltpu.make_async_copy(gb[b], agg.at[didx.at[u]], st[b]).wait()

    rbase = pl.multiple_of(s * ROWS_PT, 8)
    pltpu.sync_copy(z_ref, gb[0])
    off = 0
    for n in ZCH:
        pltpu.sync_copy(gb[0].at[pl.ds(0, n)],
                        agg.at[pl.ds(rbase + off, n)])
        off += n
    plsc.subcore_barrier()

    pltpu.sync_copy(si_ref.at[pl.ds(pl.multiple_of(wid * EPT, 8), EPT)], sidx)
    pltpu.sync_copy(di_ref.at[wid], didx)

    for u in range(2):
        ld_start(u, u)
        g_start(u, u)
    for k in range(4):
        u = k
        bA = (k + 2) % 4
        if u >= 2:
            st_wait(u - 2, bA)
        ld_start(u + 2, bA)
        g_start(u + 2, bA)
        ld_wait(u, k)
        g_wait(u, k)
        _relu_add2p(gb[k], eb[k])
        st_start(u, k)

    def grp(t, _):
        for k in range(4):
            u = t * 4 + k
            bA = (k + 2) % 4
            st_wait(u - 2, bA)
            ld_start(u + 2, bA)
            g_start(u + 2, bA)
            ld_wait(u, k)
            g_wait(u, k)
            _relu_add2p(gb[k], eb[k])
            st_start(u, k)
        return 0
    lax.fori_loop(1, 17, grp, 0)

    for u in range(68, 78):
        b = u % 4
        if u + 2 <= 77:
            bA = (u + 2) % 4
            st_wait(u - 2, bA)
            ld_start(u + 2, bA)
            g_start(u + 2, bA)
        ld_wait(u, b)
        g_wait(u, b)
        _relu_add2p(gb[b], eb[b])
        st_start(u, b)
    for u in range(76, 78):
        st_wait(u, u % 4)

    plsc.subcore_barrier()
    off = 0
    for n in ZCH:
        pltpu.sync_copy(agg.at[pl.ds(rbase + off, n)], gb[0].at[pl.ds(0, n)])
        pltpu.sync_copy(gb[0].at[pl.ds(0, n)],
                        out_ref.at[c, pl.ds(rbase + off, n)])
        off += n


def _sc_msg_pass(hm, etmp2, sidxf, didx3, zeros_pt, row_off):
    kfn = pl.kernel(
        functools.partial(_sc_msg_body, row_off=row_off),
        out_type=jax.ShapeDtypeStruct((NC, NAGG, H), F32),
        mesh=_mesh(),
        compiler_params=_SC_PARAMS,
        scratch_types=[
            pltpu.VMEM((EPT,), jnp.int32),
            pltpu.VMEM((CPT, CHUNK), jnp.int32),
            pltpu.VMEM_SHARED((NAGG, H), F32),
        ] + [pltpu.VMEM((PCH, 2 * H), F32)] * MR
          + [pltpu.VMEM((CHUNK, H), F32)] * MR
          + [pltpu.SemaphoreType.DMA] * (3 * MR),
    )
    return kfn(hm, etmp2, sidxf, didx3, zeros_pt)


def _sc_eupd_body(u1_ref, u2_ref, e3_ref, si_ref, di_ref, tsi_ref, tdi_ref,
                  e1_ref, te1_ref, *refs):
    sidx, didx, tsid, tdid = refs[0], refs[1], refs[2], refs[3]
    eb = refs[4:4 + ER]
    g1 = refs[4 + ER:4 + 2 * ER]
    g2 = refs[4 + 2 * ER:4 + 3 * ER]
    sl = refs[4 + 3 * ER:4 + 4 * ER]
    sg = refs[4 + 4 * ER:4 + 5 * ER]
    st = refs[4 + 5 * ER:4 + 6 * ER]
    c = lax.axis_index("c")
    s = lax.axis_index("s")
    wid = c * NS + s
    ebase = B // 2 + wid * HPT

    def ld_start(u, b):
        pltpu.async_copy(
            e3_ref.at[pl.ds(pl.multiple_of(ebase + u * PCH, 8), PCH)],
            eb[b], sl[b])

    def ld_wait(u, b):
        pltpu.make_async_copy(
            e3_ref.at[pl.ds(pl.multiple_of(ebase + u * PCH, 8), PCH)],
            eb[b], sl[b]).wait()

    def g_start(u, b):
        jsl = pl.ds(pl.multiple_of(u * CHUNK, 8), CHUNK)
        pltpu.async_copy(u1_ref.at[sidx.at[jsl]], g1[b], sg[b])
        pltpu.async_copy(u2_ref.at[didx.at[jsl]], g2[b], sg[b])

    def g_wait(u, b):
        jsl = pl.ds(pl.multiple_of(u * CHUNK, 8), CHUNK)
        pltpu.make_async_copy(u1_ref.at[sidx.at[jsl]], g1[b], sg[b]).wait()
        pltpu.make_async_copy(u2_ref.at[didx.at[jsl]], g2[b], sg[b]).wait()

    def st_start(u, b):
        pltpu.async_copy(
            eb[b],
            e1_ref.at[pl.ds(pl.multiple_of(wid * HPT + u * PCH, 8), PCH)],
            st[b])

    def st_wait(u, b):
        pltpu.make_async_copy(
            eb[b],
            e1_ref.at[pl.ds(pl.multiple_of(wid * HPT + u * PCH, 8), PCH)],
            st[b]).wait()

    pltpu.sync_copy(si_ref.at[pl.ds(pl.multiple_of(wid * EPT, 8), EPT)], sidx)
    pltpu.sync_copy(di_ref.at[pl.ds(pl.multiple_of(wid * EPT, 8), EPT)], didx)

    for u in range(2):
        ld_start(u, u)
        g_start(u, u)
    for k in range(4):
        u = k
        bA = (k + 2) % 4
        if u >= 2:
            st_wait(u - 2, bA)
        ld_start(u + 2, bA)
        g_start(u + 2, bA)
        ld_wait(u, k)
        g_wait(u, k)
        _relu_add3p(eb[k], g1[k], g2[k])
        st_start(u, k)

    def grp(t, _):
        for k in range(4):
            u = t * 4 + k
            bA = (k + 2) % 4
            st_wait(u - 2, bA)
            ld_start(u + 2, bA)
            g_start(u + 2, bA)
            ld_wait(u, k)
            g_wait(u, k)
            _relu_add3p(eb[k], g1[k], g2[k])
            st_start(u, k)
        return 0
    lax.fori_loop(1, 17, grp, 0)

    for u in range(68, 78):
        b = u % 4
        if u + 2 <= 77:
            bA = (u + 2) % 4
            st_wait(u - 2, bA)
            ld_start(u + 2, bA)
            g_start(u + 2, bA)
        ld_wait(u, b)
        g_wait(u, b)
        _relu_add3p(eb[b], g1[b], g2[b])
        st_start(u, b)
    for u in range(76, 78):
        st_wait(u, u % 4)

    tbase = pl.multiple_of(wid * CHUNK, 8)
    pltpu.sync_copy(tsi_ref.at[pl.ds(tbase, CHUNK)], tsid)
    pltpu.sync_copy(tdi_ref.at[pl.ds(tbase, CHUNK)], tdid)
    pltpu.sync_copy(e3_ref.at[pl.ds(pl.multiple_of(wid * PCH, 8), PCH)],
                    eb[0])
    pltpu.async_copy(u1_ref.at[tsid], g1[0], sg[0]).wait()
    pltpu.async_copy(u2_ref.at[tdid], g2[0], sg[0]).wait()
    def trow(rr, _):
        for h2 in range(2):
            for cc in range(H // 16):
                sl_ = pl.ds(cc * 16, 16)
                sp = pl.ds(h2 * H + cc * 16, 16)
                g1[0][2 * rr + h2, sl_] = jnp.maximum(
                    g1[0][2 * rr + h2, sl_] + g2[0][2 * rr + h2, sl_]
                    + eb[0][rr, sp], 0.0)
        return 0
    lax.fori_loop(0, CHUNK // 2, trow, 0)
    pltpu.sync_copy(g1[0], te1_ref.at[pl.ds(tbase, CHUNK)])


def _sc_eupd_pass(u1, u2, etmp3, sidxf, didxf, tsif, tdif):
    kfn = pl.kernel(
        _sc_eupd_body,
        out_type=(jax.ShapeDtypeStruct((EP // 2, 2 * H), F32),
                  jax.ShapeDtypeStruct((B, H), F32)),
        mesh=_mesh(),
        compiler_params=_SC_PARAMS,
        scratch_types=[
            pltpu.VMEM((EPT,), jnp.int32),
            pltpu.VMEM((EPT,), jnp.int32),
            pltpu.VMEM((CHUNK,), jnp.int32),
            pltpu.VMEM((CHUNK,), jnp.int32),
        ] + [pltpu.VMEM((PCH, 2 * H), F32)] * ER
          + [pltpu.VMEM((CHUNK, H), F32)] * (2 * ER)
          + [pltpu.SemaphoreType.DMA] * (3 * ER),
    )
    return kfn(u1, u2, etmp3, sidxf, didxf, tsif, tdif)


def _sc_tgather_body(t1_ref, t2_ref, t3_ref, t4_ref, tsi_ref, tdi_ref,
                     out_ref, tsid, tdid, gbuf, sem):
    c = lax.axis_index("c")
    s = lax.axis_index("s")
    wid = c * NS + s
    tbase = pl.multiple_of(wid * CHUNK, 8)
    pltpu.sync_copy(tsi_ref.at[pl.ds(tbase, CHUNK)], tsid)
    pltpu.sync_copy(tdi_ref.at[pl.ds(tbase, CHUNK)], tdid)
    pltpu.async_copy(t1_ref.at[tsid], gbuf, sem).wait()
    pltpu.sync_copy(gbuf, out_ref.at[0, pl.ds(tbase, CHUNK)])
    pltpu.async_copy(t2_ref.at[tdid], gbuf, sem).wait()
    pltpu.sync_copy(gbuf, out_ref.at[1, pl.ds(tbase, CHUNK)])
    pltpu.async_copy(t3_ref.at[tsid], gbuf, sem).wait()
    pltpu.sync_copy(gbuf, out_ref.at[2, pl.ds(tbase, CHUNK)])
    pltpu.async_copy(t4_ref.at[tdid], gbuf, sem).wait()
    pltpu.sync_copy(gbuf, out_ref.at[3, pl.ds(tbase, CHUNK)])


def _sc_tgather(t1, t2, t3, t4, tsif, tdif):
    kfn = pl.kernel(
        _sc_tgather_body,
        out_type=jax.ShapeDtypeStruct((4, B, H), F32),
        mesh=_mesh(),
        compiler_params=_SC_PARAMS,
        scratch_types=[
            pltpu.VMEM((CHUNK,), jnp.int32),
            pltpu.VMEM((CHUNK,), jnp.int32),
            pltpu.VMEM((CHUNK, H), F32),
            pltpu.SemaphoreType.DMA,
        ],
    )
    return kfn(t1, t2, t3, t4, tsif, tdif)


def kernel(x, edge_index, edge_attr, W_nenc, b_nenc, W_eenc, b_eenc,
           W_msg, W_eme, W_upd, W_self, W_eup, W_dec1, b_dec1, W_dec2,
           b_dec2):
    src, dst = edge_index[0, B:], edge_index[1, B:]
    tsrc, tdst = edge_index[0, :B], edge_index[1, :B]

    sidxf = jnp.concatenate([src, jnp.zeros((PADE,), jnp.int32)])
    didxf = jnp.concatenate([dst, jnp.full((PADE,), N, jnp.int32)])
    didx3 = didxf.reshape(NW, CPT, CHUNK)
    ea32 = edge_attr.reshape(E // 2, 2 * DE)
    zeros_pt = jnp.zeros((CHUNK, H), F32)

    bn = b_nenc.reshape(1, H)
    be = b_eenc.reshape(1, H)
    b1 = b_dec1.reshape(1, H)
    w2p = jnp.pad(W_dec2, ((0, 0), (0, 128 - C)))
    b2p = jnp.pad(b_dec2, (0, 128 - C)).reshape(1, 128)

    we0_1, we0_2, we0_3 = W_eup[0][:H], W_eup[0][H:2 * H], W_eup[0][2 * H:]
    we1_1, we1_2, we1_3 = W_eup[1][:H], W_eup[1][H:2 * H], W_eup[1][2 * H:]
    wd1_1, wd1_2, wd1_3 = W_dec1[:H], W_dec1[H:2 * H], W_dec1[2 * H:]

    NBLK = 2000
    h, hm0 = pl.pallas_call(
        _node_enc_body,
        grid=(N // NBLK,),
        in_specs=[
            pl.BlockSpec((NBLK, DF), lambda i: (i, 0)),
            pl.BlockSpec((DF, H), lambda i: (0, 0)),
            _b64(), _w64(0),
        ],
        out_specs=[pl.BlockSpec((NBLK, H), lambda i: (i, 0))] * 2,
        out_shape=[jax.ShapeDtypeStruct((N, H), F32)] * 2,
    )(x, W_nenc, bn, W_msg[0])

    PBLK = 2048
    etmp2_0, etmp3_0 = pl.pallas_call(
        _edge_enc_body,
        grid=(ETOT_P // 2 // PBLK,),
        in_specs=[
            pl.BlockSpec((PBLK, 2 * DE), lambda i: (i, 0)),
            pl.BlockSpec((DE, H), lambda i: (0, 0)),
            _b64(), _w64(0), _w64(1),
        ],
        out_specs=[pl.BlockSpec((PBLK, 2 * H), lambda i: (i, 0))] * 2,
        out_shape=[jax.ShapeDtypeStruct((ETOT_P // 2, 2 * H), F32)] * 2,
    )(ea32, W_eenc, be, W_eme[0], we0_3)

    part0 = _sc_msg_pass(hm0, etmp2_0, sidxf, didx3, zeros_pt, row_off=B // 2)

    h1, hu1_0, hu2_0, hm1 = pl.pallas_call(
        _node_upd0_body,
        grid=(N // NBLK,),
        in_specs=[
            pl.BlockSpec((NC, NBLK, H), lambda i: (0, i, 0)),
            pl.BlockSpec((NBLK, H), lambda i: (i, 0)),
            _w64(0), _w64(1), _w64(2), _w64(3), _w64(4),
        ],
        out_specs=[pl.BlockSpec((NBLK, H), lambda i: (i, 0))] * 4,
        out_shape=[jax.ShapeDtypeStruct((N, H), F32)] * 4,
    )(part0, h, W_upd[0], W_self[0], we0_1, we0_2, W_msg[1])

    e1, te1 = _sc_eupd_pass(hu1_0, hu2_0, etmp3_0, sidxf, didxf, tsrc, tdst)

    etmp2_1 = pl.pallas_call(
        _e2_body,
        grid=(EP // 2 // PBLK,),
        in_specs=[
            pl.BlockSpec((PBLK, 2 * H), lambda i: (i, 0)),
            _w64(0),
        ],
        out_specs=pl.BlockSpec((PBLK, 2 * H), lambda i: (i, 0)),
        out_shape=jax.ShapeDtypeStruct((EP // 2, 2 * H), F32),
    )(e1, W_eme[1])

    part1 = _sc_msg_pass(hm1, etmp2_1, sidxf, didx3, zeros_pt, row_off=0)

    hu1_1, hu2_1, hd1, hd2 = pl.pallas_call(
        _node_upd1_body,
        grid=(N // NBLK,),
        in_specs=[
            pl.BlockSpec((NC, NBLK, H), lambda i: (0, i, 0)),
            pl.BlockSpec((NBLK, H), lambda i: (i, 0)),
            _w64(0), _w64(1), _w64(2), _w64(3), _w64(4), _w64(5),
        ],
        out_specs=[pl.BlockSpec((NBLK, H), lambda i: (i, 0))] * 4,
        out_shape=[jax.ShapeDtypeStruct((N, H), F32)] * 4,
    )(part1, h1, W_upd[1], W_self[1], we1_1, we1_2, wd1_1, wd1_2)

    g = _sc_tgather(hu1_1, hu2_1, hd1, hd2, tsrc, tdst)

    TBLK = 1024
    outp = pl.pallas_call(
        _head_body,
        grid=(B // TBLK,),
        in_specs=[
            pl.BlockSpec((TBLK, H), lambda i: (i, 0)),
            pl.BlockSpec((4, TBLK, H), lambda i: (0, i, 0)),
            _w64(0), _w64(1), _b64(),
            pl.BlockSpec((H, 128), lambda i: (0, 0)),
            pl.BlockSpec((1, 128), lambda i: (0, 0)),
        ],
        out_specs=pl.BlockSpec((TBLK, 128), lambda i: (i, 0)),
        out_shape=jax.ShapeDtypeStruct((B, 128), F32),
    )(te1, g, we1_3, wd1_3, b1, w2p, b2p)

    return outp[:, :C]
